# Initial kernel scaffold; baseline (speedup 1.0000x reference)
import jax
import jax.numpy as jnp
from jax.experimental import pallas as pl


def kernel(query, reference_points, inputs, input_spatial_shapes, input_level_start_index, Wv, bv, Ws, bs, Wa, ba, Wo, bo):
    raise NotImplementedError("write your pallas kernel here")


# speedup vs baseline: 7.9583x; 7.9583x over previous
import functools
import math

import numpy as np
import jax
import jax.numpy as jnp
from jax import lax
from jax.experimental import pallas as pl
from jax.experimental.pallas import tpu as pltpu
from jax.experimental.pallas import tpu_sc as plsc

_N_HEADS = 8
_N_LEVELS = 4
_N_POINTS = 4
_C = 256
_BATCH = 4
_LEN_Q = 300
_LEN_QP = 320
_NQ_TOT = _BATCH * _LEN_QP
_SS = np.array([[128, 128], [64, 64], [32, 32], [16, 16]], dtype=np.int64)
_LS = np.array([0, 16384, 20480, 21504], dtype=np.int64)
_LEN_IN = 21760
_NW = 32
_NQPW = _NQ_TOT // _NW


def _value_body(x_ref, w_ref, b_ref, o_ref):
    o_ref[...] = jnp.dot(x_ref[...], w_ref[...],
                         preferred_element_type=jnp.float32) + b_ref[...]


def _value_proj(inputs2, Wv, bv2):
    n = inputs2.shape[0]
    blk = 640
    grid = n // blk
    return pl.pallas_call(
        _value_body,
        grid=(grid,),
        in_specs=[
            pl.BlockSpec((blk, _C), lambda i: (i, 0)),
            pl.BlockSpec((_C, _C), lambda i: (0, 0)),
            pl.BlockSpec((1, _C), lambda i: (0, 0)),
        ],
        out_specs=pl.BlockSpec((blk, _C), lambda i: (i, 0)),
        out_shape=jax.ShapeDtypeStruct((n, _C), jnp.float32),
    )(inputs2, Wv, bv2)


def _sampling_body(q_ref, rx_ref, ry_ref, ws_ref, bs_ref, wa_ref, ba_ref,
                   cc_ref, idx_ref, wgt_ref):
    b = pl.program_id(0)
    q = q_ref[...]
    off = jnp.dot(q, ws_ref[...], preferred_element_type=jnp.float32) + bs_ref[...]
    araw = jnp.dot(q, wa_ref[...], preferred_element_type=jnp.float32) + ba_ref[...]
    parts = []
    for h in range(_N_HEADS):
        a = araw[:, h * 16:(h + 1) * 16]
        m = jnp.max(a, axis=1, keepdims=True)
        e = jnp.exp(a - m)
        parts.append(e / jnp.sum(e, axis=1, keepdims=True))
    attn = jnp.concatenate(parts, axis=1)

    Wf = cc_ref[0:1, :]
    Hf = cc_ref[1:2, :]
    invWf = cc_ref[2:3, :]
    invHf = cc_ref[3:4, :]
    startf = cc_ref[4:5, :]
    hf = cc_ref[5:6, :]

    locx = rx_ref[...] + off[:, :128] * invWf
    locy = ry_ref[...] + off[:, 128:] * invHf
    x = locx * Wf - 0.5
    y = locy * Hf - 0.5
    x0 = jnp.floor(x)
    y0 = jnp.floor(y)
    x1 = x0 + 1.0
    y1 = y0 + 1.0
    wx1 = x - x0
    wx0 = 1.0 - wx1
    wy1 = y - y0
    wy0 = 1.0 - wy1

    Wi = Wf.astype(jnp.int32)
    hi = hf.astype(jnp.int32)
    basei = startf.astype(jnp.int32) + b * _LEN_IN

    corners = [(x0, y0, wx0 * wy0), (x1, y0, wx1 * wy0),
               (x0, y1, wx0 * wy1), (x1, y1, wx1 * wy1)]
    for c, (xi, yi, wb) in enumerate(corners):
        inb = (xi >= 0.) & (xi <= Wf - 1.) & (yi >= 0.) & (yi <= Hf - 1.)
        xc = jnp.clip(xi, 0., Wf - 1.).astype(jnp.int32)
        yc = jnp.clip(yi, 0., Hf - 1.).astype(jnp.int32)
        rowi = (basei + yc * Wi + xc) * _N_HEADS + hi
        idx_ref[:, c * 128:(c + 1) * 128] = rowi
        wgt_ref[:, c * 128:(c + 1) * 128] = jnp.where(inb, attn * wb, 0.0)


def _sampling(qp, rpx, rpy, Ws_p, bs_p, Wa, ba2, cc):
    return pl.pallas_call(
        _sampling_body,
        grid=(_BATCH,),
        in_specs=[
            pl.BlockSpec((_LEN_QP, _C), lambda b: (b, 0)),
            pl.BlockSpec((_LEN_QP, 128), lambda b: (b, 0)),
            pl.BlockSpec((_LEN_QP, 128), lambda b: (b, 0)),
            pl.BlockSpec((_C, _C), lambda b: (0, 0)),
            pl.BlockSpec((1, _C), lambda b: (0, 0)),
            pl.BlockSpec((_C, 128), lambda b: (0, 0)),
            pl.BlockSpec((1, 128), lambda b: (0, 0)),
            pl.BlockSpec((8, 128), lambda b: (0, 0)),
        ],
        out_specs=[
            pl.BlockSpec((_LEN_QP, 512), lambda b: (b, 0)),
            pl.BlockSpec((_LEN_QP, 512), lambda b: (b, 0)),
        ],
        out_shape=[
            jax.ShapeDtypeStruct((_NQ_TOT, 512), jnp.int32),
            jax.ShapeDtypeStruct((_NQ_TOT, 512), jnp.float32),
        ],
    )(qp, rpx, rpy, Ws_p, bs_p, Wa, ba2, cc)


def _sc_body(value_hbm, idx_hbm, wgt_hbm, out_hbm,
             idx_v, wgt_v, rows_v, outc_v, sem):
    wid = lax.axis_index("s") * 2 + lax.axis_index("c")
    base_q = wid * _NQPW
    pltpu.sync_copy(idx_hbm.at[pl.ds(base_q, _NQPW)], idx_v)
    pltpu.sync_copy(wgt_hbm.at[pl.ds(base_q, _NQPW)], wgt_v)

    def q_body(qi, carry):
        cps = [pltpu.async_copy(value_hbm.at[idx_v.at[qi, pl.ds(c * 128, 128)]], rows_v.at[c], sem)
               for c in range(4)]
        for cp in cps:
            cp.wait()

        def h_body(h, carry2):
            wvs = [wgt_v[qi, pl.ds(c * 128 + h * 16, 16)] for c in range(4)]
            lo = jnp.zeros((16,), jnp.float32)
            hi = jnp.zeros((16,), jnp.float32)
            for j in range(16):
                col = h * 16 + j
                for c in range(4):
                    w = wvs[c][j]
                    lo = lo + w * rows_v[c, col, pl.ds(0, 16)]
                    hi = hi + w * rows_v[c, col, pl.ds(16, 16)]
            outc_v[qi, pl.ds(h * 32, 16)] = lo
            outc_v[qi, pl.ds(h * 32 + 16, 16)] = hi
            return carry2

        lax.fori_loop(0, _N_HEADS, h_body, 0)
        return carry

    lax.fori_loop(0, _NQPW, q_body, 0)
    pltpu.sync_copy(outc_v, out_hbm.at[pl.ds(base_q, _NQPW)])


def _sc_gather(value2, idx3, wgt3):
    mesh = plsc.VectorSubcoreMesh(core_axis_name="c", subcore_axis_name="s")
    run = functools.partial(
        pl.kernel,
        out_type=jax.ShapeDtypeStruct((_NQ_TOT, _C), jnp.float32),
        mesh=mesh,
        scratch_types=[
            pltpu.VMEM((_NQPW, 512), jnp.int32),
            pltpu.VMEM((_NQPW, 512), jnp.float32),
            pltpu.VMEM((4, 128, 32), jnp.float32),
            pltpu.VMEM((_NQPW, _C), jnp.float32),
            pltpu.SemaphoreType.DMA,
        ],
        compiler_params=pltpu.CompilerParams(use_tc_tiling_on_sc=False),
    )(_sc_body)
    return run(value2, idx3, wgt3)


def _out_body(x_ref, w_ref, b_ref, o_ref):
    o_ref[...] = jnp.dot(x_ref[...], w_ref[...],
                         preferred_element_type=jnp.float32) + b_ref[...]


def _out_proj(acc, Wo, bo2):
    return pl.pallas_call(
        _out_body,
        grid=(1,),
        in_specs=[
            pl.BlockSpec((_NQ_TOT, _C), lambda i: (0, 0)),
            pl.BlockSpec((_C, _C), lambda i: (0, 0)),
            pl.BlockSpec((1, _C), lambda i: (0, 0)),
        ],
        out_specs=pl.BlockSpec((_NQ_TOT, _C), lambda i: (0, 0)),
        out_shape=jax.ShapeDtypeStruct((_NQ_TOT, _C), jnp.float32),
    )(acc, Wo, bo2)


def _col_consts():
    j = np.arange(128)
    l_of = (j // 4) % 4
    h_of = j // 16
    cc = np.zeros((8, 128), dtype=np.float32)
    cc[0] = _SS[l_of, 1].astype(np.float32)
    cc[1] = _SS[l_of, 0].astype(np.float32)
    cc[2] = 1.0 / cc[0]
    cc[3] = 1.0 / cc[1]
    cc[4] = _LS[l_of].astype(np.float32)
    cc[5] = h_of.astype(np.float32)
    return cc


_CC = _col_consts()


def kernel(query, reference_points, inputs, input_spatial_shapes,
           input_level_start_index, Wv, bv, Ws, bs, Wa, ba, Wo, bo):
    inputs2 = inputs.reshape(_BATCH * _LEN_IN, _C)
    qp = jnp.pad(query, ((0, 0), (0, _LEN_QP - _LEN_Q), (0, 0))).reshape(_NQ_TOT, _C)
    rpp = jnp.pad(reference_points,
                  ((0, 0), (0, _LEN_QP - _LEN_Q), (0, 0))).reshape(_NQ_TOT, 2)
    rpx = jnp.broadcast_to(rpp[:, 0:1], (_NQ_TOT, 128))
    rpy = jnp.broadcast_to(rpp[:, 1:2], (_NQ_TOT, 128))
    Ws_p = jnp.concatenate([Ws[:, 0::2], Ws[:, 1::2]], axis=1)
    bs_p = jnp.concatenate([bs[0::2], bs[1::2]]).reshape(1, _C)
    ba2 = ba.reshape(1, 128)
    bv2 = bv.reshape(1, _C)
    bo2 = bo.reshape(1, _C)

    value = _value_proj(inputs2, Wv, bv2)
    value2 = value.reshape(_BATCH * _LEN_IN * _N_HEADS, _C // _N_HEADS)
    idx, wgt = _sampling(qp, rpx, rpy, Ws_p, bs_p, Wa, ba2, jnp.asarray(_CC))
    acc = _sc_gather(value2, idx, wgt)
    out = _out_proj(acc, Wo, bo2)
    return out.reshape(_BATCH, _LEN_QP, _C)[:, :_LEN_Q]


# speedup vs baseline: 10.0062x; 1.2573x over previous
import functools
import math

import numpy as np
import jax
import jax.numpy as jnp
from jax import lax
from jax.experimental import pallas as pl
from jax.experimental.pallas import tpu as pltpu
from jax.experimental.pallas import tpu_sc as plsc

_N_HEADS = 8
_N_LEVELS = 4
_N_POINTS = 4
_C = 256
_BATCH = 4
_LEN_Q = 300
_LEN_QP = 320
_NQ_TOT = _BATCH * _LEN_QP
_SS = np.array([[128, 128], [64, 64], [32, 32], [16, 16]], dtype=np.int64)
_LS = np.array([0, 16384, 20480, 21504], dtype=np.int64)
_LEN_IN = 21760
_NW = 32
_NQPW = _NQ_TOT // _NW


def _value_body(x_ref, wl_ref, wr_ref, bl_ref, br_ref, o_ref):
    x = x_ref[...]
    blk = x.shape[0]
    lo = jnp.dot(x, wl_ref[...], preferred_element_type=jnp.float32) + bl_ref[...]
    hi = jnp.dot(x, wr_ref[...], preferred_element_type=jnp.float32) + br_ref[...]
    o_ref[:, 0] = lo.reshape(blk // 8, 8, 128)
    o_ref[:, 1] = hi.reshape(blk // 8, 8, 128)


def _value_proj(inputs2, Wv, bv):
    n = inputs2.shape[0]
    blk = 640
    grid = n // blk
    return pl.pallas_call(
        _value_body,
        grid=(grid,),
        in_specs=[
            pl.BlockSpec((blk, _C), lambda i: (i, 0)),
            pl.BlockSpec((_C, 128), lambda i: (0, 0)),
            pl.BlockSpec((_C, 128), lambda i: (0, 0)),
            pl.BlockSpec((1, 128), lambda i: (0, 0)),
            pl.BlockSpec((1, 128), lambda i: (0, 0)),
        ],
        out_specs=pl.BlockSpec((blk // 8, 2, 8, 128), lambda i: (i, 0, 0, 0)),
        out_shape=jax.ShapeDtypeStruct((n // 8, 2, 8, 128), jnp.float32),
    )(inputs2, Wv[:, :128], Wv[:, 128:], bv[:128].reshape(1, 128),
      bv[128:].reshape(1, 128))


def _sampling_body(q_ref, rx_ref, ry_ref, ws_ref, bs_ref, wa_ref, ba_ref,
                   cc_ref, idx_ref, wgt_ref):
    b = pl.program_id(0)
    q = q_ref[...]
    off = jnp.dot(q, ws_ref[...], preferred_element_type=jnp.float32) + bs_ref[...]
    araw = jnp.dot(q, wa_ref[...], preferred_element_type=jnp.float32) + ba_ref[...]
    parts = []
    for h in range(_N_HEADS):
        a = araw[:, h * 16:(h + 1) * 16]
        m = jnp.max(a, axis=1, keepdims=True)
        e = jnp.exp(a - m)
        parts.append(e / jnp.sum(e, axis=1, keepdims=True))
    attn = jnp.concatenate(parts, axis=1)

    Wf = cc_ref[0:1, :]
    Hf = cc_ref[1:2, :]
    invWf = cc_ref[2:3, :]
    invHf = cc_ref[3:4, :]
    startf = cc_ref[4:5, :]
    hf = cc_ref[5:6, :]

    locx = rx_ref[...] + off[:, :128] * invWf
    locy = ry_ref[...] + off[:, 128:] * invHf
    x = locx * Wf - 0.5
    y = locy * Hf - 0.5
    x0 = jnp.floor(x)
    y0 = jnp.floor(y)
    x1 = x0 + 1.0
    y1 = y0 + 1.0
    wx1 = x - x0
    wx0 = 1.0 - wx1
    wy1 = y - y0
    wy0 = 1.0 - wy1

    Wi = Wf.astype(jnp.int32)
    hi = hf.astype(jnp.int32)
    basei = startf.astype(jnp.int32) + b * _LEN_IN

    corners = [(x0, y0, wx0 * wy0), (x1, y0, wx1 * wy0),
               (x0, y1, wx0 * wy1), (x1, y1, wx1 * wy1)]
    for c, (xi, yi, wb) in enumerate(corners):
        inb = (xi >= 0.) & (xi <= Wf - 1.) & (yi >= 0.) & (yi <= Hf - 1.)
        xc = jnp.clip(xi, 0., Wf - 1.).astype(jnp.int32)
        yc = jnp.clip(yi, 0., Hf - 1.).astype(jnp.int32)
        r = basei + yc * Wi + xc
        rowi = ((r >> 3) << 6) + ((r & 7) << 2) + hi
        idx_ref[:, c * 128:(c + 1) * 128] = rowi
        wgt_ref[:, c * 128:(c + 1) * 128] = jnp.where(inb, attn * wb, 0.0)


def _sampling(qp, rpx, rpy, Ws_p, bs_p, Wa, ba2, cc):
    return pl.pallas_call(
        _sampling_body,
        grid=(_BATCH,),
        in_specs=[
            pl.BlockSpec((_LEN_QP, _C), lambda b: (b, 0)),
            pl.BlockSpec((_LEN_QP, 128), lambda b: (b, 0)),
            pl.BlockSpec((_LEN_QP, 128), lambda b: (b, 0)),
            pl.BlockSpec((_C, _C), lambda b: (0, 0)),
            pl.BlockSpec((1, _C), lambda b: (0, 0)),
            pl.BlockSpec((_C, 128), lambda b: (0, 0)),
            pl.BlockSpec((1, 128), lambda b: (0, 0)),
            pl.BlockSpec((8, 128), lambda b: (0, 0)),
        ],
        out_specs=[
            pl.BlockSpec((_LEN_QP, 512), lambda b: (b, 0)),
            pl.BlockSpec((_LEN_QP, 512), lambda b: (b, 0)),
        ],
        out_shape=[
            jax.ShapeDtypeStruct((_NQ_TOT, 512), jnp.int32),
            jax.ShapeDtypeStruct((_NQ_TOT, 512), jnp.float32),
        ],
    )(qp, rpx, rpy, Ws_p, bs_p, Wa, ba2, cc)


def _sc_body(value_hbm, idx_hbm, wgt_hbm, out_hbm,
             idx_v, wgt_v, rows_v, outc_v, sem0, sem1):
    wid = lax.axis_index("s") * 2 + lax.axis_index("c")
    base_q = wid * _NQPW
    sems = (sem0, sem1)
    pltpu.sync_copy(idx_hbm.at[pl.ds(base_q, _NQPW)], idx_v)
    pltpu.sync_copy(wgt_hbm.at[pl.ds(base_q, _NQPW)], wgt_v)

    def issue(qi, slot):
        for c in range(4):
            pltpu.async_copy(
                value_hbm.at[idx_v.at[qi, pl.ds(c * 128, 128)]],
                rows_v.at[slot, c], sems[slot])

    def drain(qi, slot):
        for c in range(4):
            pltpu.make_async_copy(
                value_hbm.at[idx_v.at[qi, pl.ds(c * 128, 128)]],
                rows_v.at[slot, c], sems[slot]).wait()

    def compute(qi, slot):
        def h_body(h, carry2):
            wvs = [wgt_v[qi, pl.ds(c * 128 + h * 16, 16)] for c in range(4)]
            lo = jnp.zeros((16,), jnp.float32)
            hi = jnp.zeros((16,), jnp.float32)
            for j in range(16):
                col = h * 16 + j
                for c in range(4):
                    w = wvs[c][j]
                    lo = lo + w * rows_v[slot, c, col, pl.ds(0, 16)]
                    hi = hi + w * rows_v[slot, c, col, pl.ds(16, 16)]
            outc_v[qi, pl.ds(h * 32, 16)] = lo
            outc_v[qi, pl.ds(h * 32 + 16, 16)] = hi
            return carry2

        lax.fori_loop(0, _N_HEADS, h_body, 0)

    issue(0, 0)

    def q2_body(qq, carry):
        q0 = qq * 2
        issue(q0 + 1, 1)
        drain(q0, 0)
        compute(q0, 0)

        @pl.when(qq < _NQPW // 2 - 1)
        def _():
            issue(q0 + 2, 0)

        drain(q0 + 1, 1)
        compute(q0 + 1, 1)
        return carry

    lax.fori_loop(0, _NQPW // 2, q2_body, 0)
    pltpu.sync_copy(outc_v, out_hbm.at[pl.ds(base_q, _NQPW)])


def _sc_gather(value2, idx3, wgt3):
    mesh = plsc.VectorSubcoreMesh(core_axis_name="c", subcore_axis_name="s")
    run = functools.partial(
        pl.kernel,
        out_type=jax.ShapeDtypeStruct((_NQ_TOT, _C), jnp.float32),
        mesh=mesh,
        scratch_types=[
            pltpu.VMEM((_NQPW, 512), jnp.int32),
            pltpu.VMEM((_NQPW, 512), jnp.float32),
            pltpu.VMEM((2, 4, 128, 32), jnp.float32),
            pltpu.VMEM((_NQPW, _C), jnp.float32),
            pltpu.SemaphoreType.DMA,
            pltpu.SemaphoreType.DMA,
        ],
        compiler_params=pltpu.CompilerParams(use_tc_tiling_on_sc=False),
    )(_sc_body)
    return run(value2, idx3, wgt3)


def _out_body(x_ref, w_ref, b_ref, o_ref):
    o_ref[...] = jnp.dot(x_ref[...], w_ref[...],
                         preferred_element_type=jnp.float32) + b_ref[...]


def _out_proj(acc, Wo, bo2):
    return pl.pallas_call(
        _out_body,
        grid=(1,),
        in_specs=[
            pl.BlockSpec((_NQ_TOT, _C), lambda i: (0, 0)),
            pl.BlockSpec((_C, _C), lambda i: (0, 0)),
            pl.BlockSpec((1, _C), lambda i: (0, 0)),
        ],
        out_specs=pl.BlockSpec((_NQ_TOT, _C), lambda i: (0, 0)),
        out_shape=jax.ShapeDtypeStruct((_NQ_TOT, _C), jnp.float32),
    )(acc, Wo, bo2)


def _col_consts():
    j = np.arange(128)
    l_of = (j // 4) % 4
    h_of = j // 16
    cc = np.zeros((8, 128), dtype=np.float32)
    cc[0] = _SS[l_of, 1].astype(np.float32)
    cc[1] = _SS[l_of, 0].astype(np.float32)
    cc[2] = 1.0 / cc[0]
    cc[3] = 1.0 / cc[1]
    cc[4] = _LS[l_of].astype(np.float32)
    cc[5] = ((h_of // 4) * 32 + (h_of % 4)).astype(np.float32)
    return cc


_CC = _col_consts()


def kernel(query, reference_points, inputs, input_spatial_shapes,
           input_level_start_index, Wv, bv, Ws, bs, Wa, ba, Wo, bo):
    inputs2 = inputs.reshape(_BATCH * _LEN_IN, _C)
    qp = jnp.pad(query, ((0, 0), (0, _LEN_QP - _LEN_Q), (0, 0))).reshape(_NQ_TOT, _C)
    rpp = jnp.pad(reference_points,
                  ((0, 0), (0, _LEN_QP - _LEN_Q), (0, 0))).reshape(_NQ_TOT, 2)
    rpx = jnp.broadcast_to(rpp[:, 0:1], (_NQ_TOT, 128))
    rpy = jnp.broadcast_to(rpp[:, 1:2], (_NQ_TOT, 128))
    Ws_p = jnp.concatenate([Ws[:, 0::2], Ws[:, 1::2]], axis=1)
    bs_p = jnp.concatenate([bs[0::2], bs[1::2]]).reshape(1, _C)
    ba2 = ba.reshape(1, 128)
    bo2 = bo.reshape(1, _C)

    value4 = _value_proj(inputs2, Wv, bv)
    value2 = value4.reshape(_BATCH * _LEN_IN * _N_HEADS, _C // _N_HEADS)
    idx, wgt = _sampling(qp, rpx, rpy, Ws_p, bs_p, Wa, ba2, jnp.asarray(_CC))
    acc = _sc_gather(value2, idx, wgt)
    out = _out_proj(acc, Wo, bo2)
    return out.reshape(_BATCH, _LEN_QP, _C)[:, :_LEN_Q]


# speedup vs baseline: 10.3789x; 1.0372x over previous
import functools

import numpy as np
import jax
import jax.numpy as jnp
from jax import lax
from jax.experimental import pallas as pl
from jax.experimental.pallas import tpu as pltpu
from jax.experimental.pallas import tpu_sc as plsc

_N_HEADS = 8
_C = 256
_BATCH = 4
_LEN_Q = 300
_LEN_QP = 320
_NQ_TOT = _BATCH * _LEN_QP
_NQ_HALF = _NQ_TOT // 2
_SS = np.array([[128, 128], [64, 64], [32, 32], [16, 16]], dtype=np.int64)
_LS = np.array([0, 16384, 20480, 21504], dtype=np.int64)
_LEN_IN = 21760
_NW = 32
_NQPW = _NQ_HALF // _NW


def _rtne_bf16_bits(v):
    bits = lax.bitcast_convert_type(v, jnp.int32)
    r = bits + 0x7FFF + ((bits >> 16) & 1)
    return (r >> 16) & 0xFFFF


def _value_body(x_ref, wl_ref, wr_ref, bl_ref, br_ref, o_ref):
    x = x_ref[...].astype(jnp.bfloat16)
    wl = wl_ref[...].astype(jnp.bfloat16)
    wr = wr_ref[...].astype(jnp.bfloat16)
    lo = jnp.dot(x, wl, preferred_element_type=jnp.float32) + bl_ref[...]
    hi = jnp.dot(x, wr, preferred_element_type=jnp.float32) + br_ref[...]
    o_ref[...] = _rtne_bf16_bits(lo) | (_rtne_bf16_bits(hi) << 16)


def _value_proj(inputs2, Wvl, Wvr, bvl, bvr):
    n = inputs2.shape[0]
    blk = 1280
    grid = n // blk
    return pl.pallas_call(
        _value_body,
        grid=(grid,),
        in_specs=[
            pl.BlockSpec((blk, _C), lambda i: (i, 0)),
            pl.BlockSpec((_C, 128), lambda i: (0, 0)),
            pl.BlockSpec((_C, 128), lambda i: (0, 0)),
            pl.BlockSpec((1, 128), lambda i: (0, 0)),
            pl.BlockSpec((1, 128), lambda i: (0, 0)),
        ],
        out_specs=pl.BlockSpec((blk, 128), lambda i: (i, 0)),
        out_shape=jax.ShapeDtypeStruct((n, 128), jnp.int32),
    )(inputs2, Wvl, Wvr, bvl, bvr)


def _sampling_body(q_ref, rx_ref, ry_ref, ws_ref, bs_ref, wa_ref, ba_ref,
                   cc_ref, idx_ref, wgt_ref):
    b = pl.program_id(0)
    q = q_ref[...]
    off = jnp.dot(q, ws_ref[...], preferred_element_type=jnp.float32) + bs_ref[...]
    araw = jnp.dot(q, wa_ref[...], preferred_element_type=jnp.float32) + ba_ref[...]
    parts = []
    for h in range(_N_HEADS):
        a = araw[:, h * 16:(h + 1) * 16]
        m = jnp.max(a, axis=1, keepdims=True)
        e = jnp.exp(a - m)
        parts.append(e / jnp.sum(e, axis=1, keepdims=True))
    attn = jnp.concatenate(parts, axis=1)

    Wf = cc_ref[0:1, :]
    Hf = cc_ref[1:2, :]
    invWf = cc_ref[2:3, :]
    invHf = cc_ref[3:4, :]
    startf = cc_ref[4:5, :]
    hf = cc_ref[5:6, :]

    locx = rx_ref[...] + off[:, :128] * invWf
    locy = ry_ref[...] + off[:, 128:] * invHf
    x = locx * Wf - 0.5
    y = locy * Hf - 0.5
    x0 = jnp.floor(x)
    y0 = jnp.floor(y)
    x1 = x0 + 1.0
    y1 = y0 + 1.0
    wx1 = x - x0
    wx0 = 1.0 - wx1
    wy1 = y - y0
    wy0 = 1.0 - wy1

    Wi = Wf.astype(jnp.int32)
    hi = hf.astype(jnp.int32)
    basei = startf.astype(jnp.int32) + (b & 1) * _LEN_IN

    corners = [(x0, y0, wx0 * wy0), (x1, y0, wx1 * wy0),
               (x0, y1, wx0 * wy1), (x1, y1, wx1 * wy1)]
    for c, (xi, yi, wb) in enumerate(corners):
        inb = (xi >= 0.) & (xi <= Wf - 1.) & (yi >= 0.) & (yi <= Hf - 1.)
        xc = jnp.clip(xi, 0., Wf - 1.).astype(jnp.int32)
        yc = jnp.clip(yi, 0., Hf - 1.).astype(jnp.int32)
        rowi = (basei + yc * Wi + xc) * _N_HEADS + hi
        idx_ref[:, c] = rowi.reshape(_LEN_QP // 8, 8, 128)
        wgt_ref[:, c] = jnp.where(inb, attn * wb, 0.0).reshape(_LEN_QP // 8, 8, 128)


def _sampling(qp, rpx, rpy, Ws_p, bs_p, Wa, ba2, cc):
    return pl.pallas_call(
        _sampling_body,
        grid=(_BATCH,),
        in_specs=[
            pl.BlockSpec((_LEN_QP, _C), lambda b: (b, 0)),
            pl.BlockSpec((_LEN_QP, 128), lambda b: (b, 0)),
            pl.BlockSpec((_LEN_QP, 128), lambda b: (b, 0)),
            pl.BlockSpec((_C, _C), lambda b: (0, 0)),
            pl.BlockSpec((1, _C), lambda b: (0, 0)),
            pl.BlockSpec((_C, 128), lambda b: (0, 0)),
            pl.BlockSpec((1, 128), lambda b: (0, 0)),
            pl.BlockSpec((8, 128), lambda b: (0, 0)),
        ],
        out_specs=[
            pl.BlockSpec((_LEN_QP // 8, 4, 8, 128), lambda b: (b, 0, 0, 0)),
            pl.BlockSpec((_LEN_QP // 8, 4, 8, 128), lambda b: (b, 0, 0, 0)),
        ],
        out_shape=[
            jax.ShapeDtypeStruct((_NQ_TOT // 8, 4, 8, 128), jnp.int32),
            jax.ShapeDtypeStruct((_NQ_TOT // 8, 4, 8, 128), jnp.float32),
        ],
    )(qp, rpx, rpy, Ws_p, bs_p, Wa, ba2, cc)


def _sc_body(value_hbm, idx_hbm, wgt_hbm, out_hbm,
             idx_v, wgt_v, rows_v, outc_v, sem0, sem1):
    wid = lax.axis_index("s") * 2 + lax.axis_index("c")
    base_q = wid * _NQPW
    base_t = base_q >> 3
    off = base_q - (base_t << 3)
    sems = (sem0, sem1)
    pltpu.sync_copy(idx_hbm.at[pl.ds(base_t, 3)], idx_v)
    pltpu.sync_copy(wgt_hbm.at[pl.ds(base_t, 3)], wgt_v)

    def issue(qi, slot):
        qi = qi + off
        t, s = qi >> 3, qi & 7
        for c in range(4):
            pltpu.async_copy(
                value_hbm.at[idx_v.at[t, c, s]],
                rows_v.at[slot, c], sems[slot])

    def drain(qi, slot):
        qi = qi + off
        t, s = qi >> 3, qi & 7
        for c in range(4):
            pltpu.make_async_copy(
                value_hbm.at[idx_v.at[t, c, s]],
                rows_v.at[slot, c], sems[slot]).wait()

    def compute(qi, slot):
        qo = qi + off
        t, s = qo >> 3, qo & 7

        def h_body(h, carry2):
            wvs = [wgt_v[t, c, s, pl.ds(h * 16, 16)] for c in range(4)]
            lo = jnp.zeros((16,), jnp.float32)
            hi = jnp.zeros((16,), jnp.float32)
            for j in range(16):
                col = h * 16 + j
                for c in range(4):
                    w = wvs[c][j]
                    w32 = rows_v[slot, c, col, ...]
                    lov = plsc.bitcast(w32 << 16, jnp.float32)
                    hiv = plsc.bitcast(w32, jnp.float32)
                    lo = lo + w * lov
                    hi = hi + w * hiv
            outc_v[qi, pl.ds(h * 32, 16)] = lo
            outc_v[qi, pl.ds(h * 32 + 16, 16)] = hi
            return carry2

        lax.fori_loop(0, _N_HEADS, h_body, 0)

    issue(0, 0)

    def q2_body(qq, carry):
        q0 = qq * 2
        issue(q0 + 1, 1)
        drain(q0, 0)
        compute(q0, 0)

        @pl.when(qq < _NQPW // 2 - 1)
        def _():
            issue(q0 + 2, 0)

        drain(q0 + 1, 1)
        compute(q0 + 1, 1)
        return carry

    lax.fori_loop(0, _NQPW // 2, q2_body, 0)
    pltpu.sync_copy(outc_v, out_hbm.at[pl.ds(base_q, _NQPW)])


def _sc_gather(value2, idx4, wgt4):
    mesh = plsc.VectorSubcoreMesh(core_axis_name="c", subcore_axis_name="s")
    run = functools.partial(
        pl.kernel,
        out_type=jax.ShapeDtypeStruct((_NQ_HALF, _C), jnp.float32),
        mesh=mesh,
        scratch_types=[
            pltpu.VMEM((3, 4, 8, 128), jnp.int32),
            pltpu.VMEM((3, 4, 8, 128), jnp.float32),
            pltpu.VMEM((2, 4, 128, 16), jnp.int32),
            pltpu.VMEM((_NQPW, _C), jnp.float32),
            pltpu.SemaphoreType.DMA,
            pltpu.SemaphoreType.DMA,
        ],
        compiler_params=pltpu.CompilerParams(use_tc_tiling_on_sc=False,
                                             needs_layout_passes=False),
    )(_sc_body)
    return run(value2, idx4, wgt4)


def _out_body(x_ref, w_ref, b_ref, o_ref):
    o_ref[...] = jnp.dot(x_ref[...], w_ref[...],
                         preferred_element_type=jnp.float32) + b_ref[...]


def _out_proj(acc, Wo, bo2):
    return pl.pallas_call(
        _out_body,
        grid=(1,),
        in_specs=[
            pl.BlockSpec((_NQ_TOT, _C), lambda i: (0, 0)),
            pl.BlockSpec((_C, _C), lambda i: (0, 0)),
            pl.BlockSpec((1, _C), lambda i: (0, 0)),
        ],
        out_specs=pl.BlockSpec((_NQ_TOT, _C), lambda i: (0, 0)),
        out_shape=jax.ShapeDtypeStruct((_NQ_TOT, _C), jnp.float32),
    )(acc, Wo, bo2)


def _col_consts():
    j = np.arange(128)
    l_of = (j // 4) % 4
    h_of = j // 16
    cc = np.zeros((8, 128), dtype=np.float32)
    cc[0] = _SS[l_of, 1].astype(np.float32)
    cc[1] = _SS[l_of, 0].astype(np.float32)
    cc[2] = 1.0 / cc[0]
    cc[3] = 1.0 / cc[1]
    cc[4] = _LS[l_of].astype(np.float32)
    cc[5] = h_of.astype(np.float32)
    return cc


_CC = _col_consts()


def kernel(query, reference_points, inputs, input_spatial_shapes,
           input_level_start_index, Wv, bv, Ws, bs, Wa, ba, Wo, bo):
    inputs2 = inputs.reshape(_BATCH * _LEN_IN, _C)
    qp = jnp.pad(query, ((0, 0), (0, _LEN_QP - _LEN_Q), (0, 0))).reshape(_NQ_TOT, _C)
    rpp = jnp.pad(reference_points,
                  ((0, 0), (0, _LEN_QP - _LEN_Q), (0, 0))).reshape(_NQ_TOT, 2)
    rpx = jnp.broadcast_to(rpp[:, 0:1], (_NQ_TOT, 128))
    rpy = jnp.broadcast_to(rpp[:, 1:2], (_NQ_TOT, 128))
    Ws_p = jnp.concatenate([Ws[:, 0::2], Ws[:, 1::2]], axis=1)
    bs_p = jnp.concatenate([bs[0::2], bs[1::2]]).reshape(1, _C)
    ba2 = ba.reshape(1, 128)
    bo2 = bo.reshape(1, _C)
    j = np.arange(128)
    cols_lo = (j // 16) * 32 + (j % 16)
    cols_hi = cols_lo + 16
    Wvl, Wvr = Wv[:, cols_lo], Wv[:, cols_hi]
    bvl = bv[cols_lo].reshape(1, 128)
    bvr = bv[cols_hi].reshape(1, 128)

    half_rows = _BATCH * _LEN_IN // 2
    idx, wgt = _sampling(qp, rpx, rpy, Ws_p, bs_p, Wa, ba2, jnp.asarray(_CC))
    packed_a = _value_proj(inputs2[:half_rows], Wvl, Wvr, bvl, bvr)
    acc_a = _sc_gather(packed_a.reshape(half_rows * _N_HEADS, 16),
                       idx[:_NQ_HALF // 8], wgt[:_NQ_HALF // 8])
    packed_b = _value_proj(inputs2[half_rows:], Wvl, Wvr, bvl, bvr)
    acc_b = _sc_gather(packed_b.reshape(half_rows * _N_HEADS, 16),
                       idx[_NQ_HALF // 8:], wgt[_NQ_HALF // 8:])
    acc = jnp.concatenate([acc_a, acc_b], axis=0)
    out = _out_proj(acc, Wo, bo2)
    return out.reshape(_BATCH, _LEN_QP, _C)[:, :_LEN_Q]


# speedup vs baseline: 15.9865x; 1.5403x over previous
import functools

import numpy as np
import jax
import jax.numpy as jnp
from jax import lax
from jax.experimental import pallas as pl
from jax.experimental.pallas import tpu as pltpu
from jax.experimental.pallas import tpu_sc as plsc

_N_HEADS = 8
_C = 256
_BATCH = 4
_LEN_Q = 300
_LEN_QP = 320
_NQ_TOT = _BATCH * _LEN_QP
_SS = np.array([[128, 128], [64, 64], [32, 32], [16, 16]], dtype=np.int64)
_LS = np.array([0, 16384, 20480, 21504], dtype=np.int64)
_LEN_IN = 21760
_NW = 32
_NQPW = _NQ_TOT // _NW


def _rtne_bf16_bits(v):
    bits = lax.bitcast_convert_type(v, jnp.int32)
    r = bits + 0x7FFF + ((bits >> 16) & 1)
    return (r >> 16) & 0xFFFF


def _value_body(x_ref, wl_ref, wr_ref, bl_ref, br_ref, o_ref):
    x = x_ref[...].astype(jnp.bfloat16)
    wl = wl_ref[...].astype(jnp.bfloat16)
    wr = wr_ref[...].astype(jnp.bfloat16)
    lo = jnp.dot(x, wl, preferred_element_type=jnp.float32) + bl_ref[...]
    hi = jnp.dot(x, wr, preferred_element_type=jnp.float32) + br_ref[...]
    o_ref[...] = _rtne_bf16_bits(lo) | (_rtne_bf16_bits(hi) << 16)


def _value_proj(inputs2, Wvl, Wvr, bvl, bvr):
    n = inputs2.shape[0]
    blk = 4352
    grid = n // blk
    return pl.pallas_call(
        _value_body,
        grid=(grid,),
        in_specs=[
            pl.BlockSpec((blk, _C), lambda i: (i, 0)),
            pl.BlockSpec((_C, 128), lambda i: (0, 0)),
            pl.BlockSpec((_C, 128), lambda i: (0, 0)),
            pl.BlockSpec((1, 128), lambda i: (0, 0)),
            pl.BlockSpec((1, 128), lambda i: (0, 0)),
        ],
        out_specs=pl.BlockSpec((blk, 128), lambda i: (i, 0)),
        out_shape=jax.ShapeDtypeStruct((n, 128), jnp.int32),
    )(inputs2, Wvl, Wvr, bvl, bvr)


def _sampling_body(q_ref, rx_ref, ry_ref, ws_ref, bs_ref, wa_ref, ba_ref,
                   cc_ref, idx_ref, wgt_ref):
    b = pl.program_id(0)
    q = q_ref[...]
    off = jnp.dot(q, ws_ref[...], preferred_element_type=jnp.float32) + bs_ref[...]
    araw = jnp.dot(q, wa_ref[...], preferred_element_type=jnp.float32) + ba_ref[...]
    parts = []
    for h in range(_N_HEADS):
        a = araw[:, h * 16:(h + 1) * 16]
        m = jnp.max(a, axis=1, keepdims=True)
        e = jnp.exp(a - m)
        parts.append(e / jnp.sum(e, axis=1, keepdims=True))
    attn = jnp.concatenate(parts, axis=1)

    Wf = cc_ref[0:1, :]
    Hf = cc_ref[1:2, :]
    invWf = cc_ref[2:3, :]
    invHf = cc_ref[3:4, :]
    startf = cc_ref[4:5, :]
    hf = cc_ref[5:6, :]

    locx = rx_ref[...] + off[:, :128] * invWf
    locy = ry_ref[...] + off[:, 128:] * invHf
    x = locx * Wf - 0.5
    y = locy * Hf - 0.5
    x0 = jnp.floor(x)
    y0 = jnp.floor(y)
    x1 = x0 + 1.0
    y1 = y0 + 1.0
    wx1 = x - x0
    wx0 = 1.0 - wx1
    wy1 = y - y0
    wy0 = 1.0 - wy1

    Wi = Wf.astype(jnp.int32)
    hi = hf.astype(jnp.int32)
    basei = startf.astype(jnp.int32) + b * _LEN_IN

    corners = [(x0, y0, wx0 * wy0), (x1, y0, wx1 * wy0),
               (x0, y1, wx0 * wy1), (x1, y1, wx1 * wy1)]
    for c, (xi, yi, wb) in enumerate(corners):
        inb = (xi >= 0.) & (xi <= Wf - 1.) & (yi >= 0.) & (yi <= Hf - 1.)
        xc = jnp.clip(xi, 0., Wf - 1.).astype(jnp.int32)
        yc = jnp.clip(yi, 0., Hf - 1.).astype(jnp.int32)
        rowi = (basei + yc * Wi + xc) * _N_HEADS + hi
        idx_ref[:, c] = rowi.reshape(_LEN_QP // 8, 8, 128)
        wgt_ref[:, c] = jnp.where(inb, attn * wb, 0.0).reshape(_LEN_QP // 8, 8, 128)


def _sampling(qp, rpx, rpy, Ws_p, bs_p, Wa, ba2, cc):
    return pl.pallas_call(
        _sampling_body,
        grid=(_BATCH,),
        in_specs=[
            pl.BlockSpec((_LEN_QP, _C), lambda b: (b, 0)),
            pl.BlockSpec((_LEN_QP, 128), lambda b: (b, 0)),
            pl.BlockSpec((_LEN_QP, 128), lambda b: (b, 0)),
            pl.BlockSpec((_C, _C), lambda b: (0, 0)),
            pl.BlockSpec((1, _C), lambda b: (0, 0)),
            pl.BlockSpec((_C, 128), lambda b: (0, 0)),
            pl.BlockSpec((1, 128), lambda b: (0, 0)),
            pl.BlockSpec((8, 128), lambda b: (0, 0)),
        ],
        out_specs=[
            pl.BlockSpec((_LEN_QP // 8, 4, 8, 128), lambda b: (b, 0, 0, 0)),
            pl.BlockSpec((_LEN_QP // 8, 4, 8, 128), lambda b: (b, 0, 0, 0)),
        ],
        out_shape=[
            jax.ShapeDtypeStruct((_NQ_TOT // 8, 4, 8, 128), jnp.int32),
            jax.ShapeDtypeStruct((_NQ_TOT // 8, 4, 8, 128), jnp.float32),
        ],
    )(qp, rpx, rpy, Ws_p, bs_p, Wa, ba2, cc)


def _sc_body(value_hbm, idx_hbm, wgt_hbm, out_hbm,
             idx_v, wgt_v, rows_v, outc_v, sem0, sem1):
    wid = lax.axis_index("s") * 2 + lax.axis_index("c")
    base_q = wid * _NQPW
    base_t = wid * (_NQPW // 8)
    sems = (sem0, sem1)
    pltpu.sync_copy(idx_hbm.at[pl.ds(base_t, _NQPW // 8)], idx_v)
    pltpu.sync_copy(wgt_hbm.at[pl.ds(base_t, _NQPW // 8)], wgt_v)

    def issue(qi, slot):
        t, s = qi >> 3, qi & 7
        for c in range(4):
            pltpu.async_copy(
                value_hbm.at[idx_v.at[t, c, s]],
                rows_v.at[slot, c], sems[slot])

    def drain(qi, slot):
        t, s = qi >> 3, qi & 7
        for c in range(4):
            pltpu.make_async_copy(
                value_hbm.at[idx_v.at[t, c, s]],
                rows_v.at[slot, c], sems[slot]).wait()

    def compute(qi, slot):
        t, s = qi >> 3, qi & 7

        def h_body(h, carry2):
            wvs = [wgt_v[t, c, s, pl.ds(h * 16, 16)] for c in range(4)]
            lo = jnp.zeros((16,), jnp.float32)
            hi = jnp.zeros((16,), jnp.float32)
            for j in range(16):
                col = h * 16 + j
                for c in range(4):
                    w = wvs[c][j]
                    w32 = rows_v[slot, c, col, ...]
                    lov = plsc.bitcast(w32 << 16, jnp.float32)
                    hiv = plsc.bitcast(w32, jnp.float32)
                    lo = lo + w * lov
                    hi = hi + w * hiv
            outc_v[qi, pl.ds(h * 32, 16)] = lo
            outc_v[qi, pl.ds(h * 32 + 16, 16)] = hi
            return carry2

        lax.fori_loop(0, _N_HEADS, h_body, 0)

    issue(0, 0)

    def q2_body(qq, carry):
        q0 = qq * 2
        issue(q0 + 1, 1)
        drain(q0, 0)
        compute(q0, 0)

        @pl.when(qq < _NQPW // 2 - 1)
        def _():
            issue(q0 + 2, 0)

        drain(q0 + 1, 1)
        compute(q0 + 1, 1)
        return carry

    lax.fori_loop(0, _NQPW // 2, q2_body, 0)
    pltpu.sync_copy(outc_v, out_hbm.at[pl.ds(base_q, _NQPW)])


def _sc_gather(value2, idx4, wgt4):
    mesh = plsc.VectorSubcoreMesh(core_axis_name="c", subcore_axis_name="s")
    run = functools.partial(
        pl.kernel,
        out_type=jax.ShapeDtypeStruct((_NQ_TOT, _C), jnp.float32),
        mesh=mesh,
        scratch_types=[
            pltpu.VMEM((_NQPW // 8, 4, 8, 128), jnp.int32),
            pltpu.VMEM((_NQPW // 8, 4, 8, 128), jnp.float32),
            pltpu.VMEM((2, 4, 128, 16), jnp.int32),
            pltpu.VMEM((_NQPW, _C), jnp.float32),
            pltpu.SemaphoreType.DMA,
            pltpu.SemaphoreType.DMA,
        ],
        compiler_params=pltpu.CompilerParams(use_tc_tiling_on_sc=False,
                                             needs_layout_passes=False),
    )(_sc_body)
    return run(value2, idx4, wgt4)


def _out_body(x_ref, w_ref, b_ref, o_ref):
    o_ref[...] = jnp.dot(x_ref[...], w_ref[...],
                         preferred_element_type=jnp.float32) + b_ref[...]


def _out_proj(acc, Wo, bo2):
    return pl.pallas_call(
        _out_body,
        grid=(1,),
        in_specs=[
            pl.BlockSpec((_NQ_TOT, _C), lambda i: (0, 0)),
            pl.BlockSpec((_C, _C), lambda i: (0, 0)),
            pl.BlockSpec((1, _C), lambda i: (0, 0)),
        ],
        out_specs=pl.BlockSpec((_NQ_TOT, _C), lambda i: (0, 0)),
        out_shape=jax.ShapeDtypeStruct((_NQ_TOT, _C), jnp.float32),
    )(acc, Wo, bo2)


def _col_consts():
    j = np.arange(128)
    l_of = (j // 4) % 4
    h_of = j // 16
    cc = np.zeros((8, 128), dtype=np.float32)
    cc[0] = _SS[l_of, 1].astype(np.float32)
    cc[1] = _SS[l_of, 0].astype(np.float32)
    cc[2] = 1.0 / cc[0]
    cc[3] = 1.0 / cc[1]
    cc[4] = _LS[l_of].astype(np.float32)
    cc[5] = h_of.astype(np.float32)
    return cc


_CC = _col_consts()


def kernel(query, reference_points, inputs, input_spatial_shapes,
           input_level_start_index, Wv, bv, Ws, bs, Wa, ba, Wo, bo):
    inputs2 = inputs.reshape(_BATCH * _LEN_IN, _C)
    qp = jnp.pad(query, ((0, 0), (0, _LEN_QP - _LEN_Q), (0, 0))).reshape(_NQ_TOT, _C)
    rpp = jnp.pad(reference_points,
                  ((0, 0), (0, _LEN_QP - _LEN_Q), (0, 0))).reshape(_NQ_TOT, 2)
    rpx = jnp.broadcast_to(rpp[:, 0:1], (_NQ_TOT, 128))
    rpy = jnp.broadcast_to(rpp[:, 1:2], (_NQ_TOT, 128))
    Ws_p = jnp.concatenate([Ws[:, 0::2], Ws[:, 1::2]], axis=1)
    bs_p = jnp.concatenate([bs[0::2], bs[1::2]]).reshape(1, _C)
    ba2 = ba.reshape(1, 128)
    bo2 = bo.reshape(1, _C)
    j = np.arange(128)
    cols_lo = (j // 16) * 32 + (j % 16)
    cols_hi = cols_lo + 16
    Wvl, Wvr = Wv[:, cols_lo], Wv[:, cols_hi]
    bvl = bv[cols_lo].reshape(1, 128)
    bvr = bv[cols_hi].reshape(1, 128)

    idx, wgt = _sampling(qp, rpx, rpy, Ws_p, bs_p, Wa, ba2, jnp.asarray(_CC))
    packed = _value_proj(inputs2, Wvl, Wvr, bvl, bvr)
    value2 = packed.reshape(_BATCH * _LEN_IN * _N_HEADS, 16)
    acc = _sc_gather(value2, idx, wgt)
    out = _out_proj(acc, Wo, bo2)
    return out.reshape(_BATCH, _LEN_QP, _C)[:, :_LEN_Q]


# speedup vs baseline: 17.2332x; 1.0780x over previous
import functools

import numpy as np
import jax
import jax.numpy as jnp
from jax import lax
from jax.experimental import pallas as pl
from jax.experimental.pallas import tpu as pltpu
from jax.experimental.pallas import tpu_sc as plsc

_N_HEADS = 8
_C = 256
_BATCH = 4
_LEN_Q = 300
_LEN_QP = 320
_NQ_TOT = _BATCH * _LEN_QP
_SS = np.array([[128, 128], [64, 64], [32, 32], [16, 16]], dtype=np.int64)
_LS = np.array([0, 16384, 20480, 21504], dtype=np.int64)
_LEN_IN = 21760
_NW = 32
_NQPW = _NQ_TOT // _NW


def _rtne_bf16_bits(v):
    bits = lax.bitcast_convert_type(v, jnp.int32)
    r = bits + 0x7FFF + ((bits >> 16) & 1)
    return (r >> 16) & 0xFFFF


def _value_body(x_ref, wl_ref, wr_ref, bl_ref, br_ref, o_ref):
    x = x_ref[...].astype(jnp.bfloat16)
    wl = wl_ref[...].astype(jnp.bfloat16)
    wr = wr_ref[...].astype(jnp.bfloat16)
    lo = jnp.dot(x, wl, preferred_element_type=jnp.float32) + bl_ref[...]
    hi = jnp.dot(x, wr, preferred_element_type=jnp.float32) + br_ref[...]
    o_ref[...] = _rtne_bf16_bits(lo) | (_rtne_bf16_bits(hi) << 16)


def _value_proj(inputs2, Wvl, Wvr, bvl, bvr):
    n = inputs2.shape[0]
    blk = 8704
    grid = n // blk
    return pl.pallas_call(
        _value_body,
        grid=(grid,),
        in_specs=[
            pl.BlockSpec((blk, _C), lambda i: (i, 0)),
            pl.BlockSpec((_C, 128), lambda i: (0, 0)),
            pl.BlockSpec((_C, 128), lambda i: (0, 0)),
            pl.BlockSpec((1, 128), lambda i: (0, 0)),
            pl.BlockSpec((1, 128), lambda i: (0, 0)),
        ],
        out_specs=pl.BlockSpec((blk, 128), lambda i: (i, 0)),
        out_shape=jax.ShapeDtypeStruct((n, 128), jnp.int32),
    )(inputs2, Wvl, Wvr, bvl, bvr)


def _sampling_body(q_ref, rx_ref, ry_ref, ws_ref, bs_ref, wa_ref, ba_ref,
                   cc_ref, idx_ref, wgt_ref):
    q = q_ref[...]
    off = jnp.dot(q, ws_ref[...], preferred_element_type=jnp.float32) + bs_ref[...]
    araw = jnp.dot(q, wa_ref[...], preferred_element_type=jnp.float32) + ba_ref[...]
    parts = []
    for h in range(_N_HEADS):
        a = araw[:, h * 16:(h + 1) * 16]
        m = jnp.max(a, axis=1, keepdims=True)
        e = jnp.exp(a - m)
        parts.append(e / jnp.sum(e, axis=1, keepdims=True))
    attn = jnp.concatenate(parts, axis=1)

    Wf = cc_ref[0:1, :]
    Hf = cc_ref[1:2, :]
    invWf = cc_ref[2:3, :]
    invHf = cc_ref[3:4, :]
    startf = cc_ref[4:5, :]
    hf = cc_ref[5:6, :]

    locx = rx_ref[...] + off[:, :128] * invWf
    locy = ry_ref[...] + off[:, 128:] * invHf
    x = locx * Wf - 0.5
    y = locy * Hf - 0.5
    x0 = jnp.floor(x)
    y0 = jnp.floor(y)
    x1 = x0 + 1.0
    y1 = y0 + 1.0
    wx1 = x - x0
    wx0 = 1.0 - wx1
    wy1 = y - y0
    wy0 = 1.0 - wy1

    Wi = Wf.astype(jnp.int32)
    hi = hf.astype(jnp.int32)
    bi = lax.broadcasted_iota(jnp.int32, (_NQ_TOT, 128), 0) // _LEN_QP
    basei = startf.astype(jnp.int32) + bi * _LEN_IN

    corners = [(x0, y0, wx0 * wy0), (x1, y0, wx1 * wy0),
               (x0, y1, wx0 * wy1), (x1, y1, wx1 * wy1)]
    for c, (xi, yi, wb) in enumerate(corners):
        inb = (xi >= 0.) & (xi <= Wf - 1.) & (yi >= 0.) & (yi <= Hf - 1.)
        xc = jnp.clip(xi, 0., Wf - 1.).astype(jnp.int32)
        yc = jnp.clip(yi, 0., Hf - 1.).astype(jnp.int32)
        rowi = (basei + yc * Wi + xc) * _N_HEADS + hi
        idx_ref[:, c] = rowi.reshape(_NQ_TOT // 8, 8, 128)
        wgt_ref[:, c] = jnp.where(inb, attn * wb, 0.0).reshape(_NQ_TOT // 8, 8, 128)


def _sampling(qp, rpx, rpy, Ws_p, bs_p, Wa, ba2, cc):
    return pl.pallas_call(
        _sampling_body,
        grid=(1,),
        in_specs=[
            pl.BlockSpec((_NQ_TOT, _C), lambda b: (0, 0)),
            pl.BlockSpec((_NQ_TOT, 128), lambda b: (0, 0)),
            pl.BlockSpec((_NQ_TOT, 128), lambda b: (0, 0)),
            pl.BlockSpec((_C, _C), lambda b: (0, 0)),
            pl.BlockSpec((1, _C), lambda b: (0, 0)),
            pl.BlockSpec((_C, 128), lambda b: (0, 0)),
            pl.BlockSpec((1, 128), lambda b: (0, 0)),
            pl.BlockSpec((8, 128), lambda b: (0, 0)),
        ],
        out_specs=[
            pl.BlockSpec((_NQ_TOT // 8, 4, 8, 128), lambda b: (0, 0, 0, 0)),
            pl.BlockSpec((_NQ_TOT // 8, 4, 8, 128), lambda b: (0, 0, 0, 0)),
        ],
        out_shape=[
            jax.ShapeDtypeStruct((_NQ_TOT // 8, 4, 8, 128), jnp.int32),
            jax.ShapeDtypeStruct((_NQ_TOT // 8, 4, 8, 128), jnp.float32),
        ],
    )(qp, rpx, rpy, Ws_p, bs_p, Wa, ba2, cc)


def _sc_body(value_hbm, idx_hbm, wgt_hbm, out_hbm,
             idx_v, wgt_v, rows_v, outc_v, sem0, sem1):
    wid = lax.axis_index("s") * 2 + lax.axis_index("c")
    base_q = wid * _NQPW
    base_t = wid * (_NQPW // 8)
    sems = (sem0, sem1)
    pltpu.sync_copy(idx_hbm.at[pl.ds(base_t, _NQPW // 8)], idx_v)
    pltpu.sync_copy(wgt_hbm.at[pl.ds(base_t, _NQPW // 8)], wgt_v)

    def issue(qi, slot):
        t, s = qi >> 3, qi & 7
        for c in range(4):
            pltpu.async_copy(
                value_hbm.at[idx_v.at[t, c, s]],
                rows_v.at[slot, c], sems[slot])

    def drain(qi, slot):
        t, s = qi >> 3, qi & 7
        for c in range(4):
            pltpu.make_async_copy(
                value_hbm.at[idx_v.at[t, c, s]],
                rows_v.at[slot, c], sems[slot]).wait()

    def compute(qi, slot):
        t, s = qi >> 3, qi & 7

        def h_body(h, carry2):
            wvs = [wgt_v[t, c, s, pl.ds(h * 16, 16)] for c in range(4)]
            lo = jnp.zeros((16,), jnp.float32)
            hi = jnp.zeros((16,), jnp.float32)
            for j in range(16):
                col = h * 16 + j
                for c in range(4):
                    w = wvs[c][j]
                    w32 = rows_v[slot, c, col, ...]
                    lov = plsc.bitcast(w32 << 16, jnp.float32)
                    hiv = plsc.bitcast(w32, jnp.float32)
                    lo = lo + w * lov
                    hi = hi + w * hiv
            outc_v[qi, pl.ds(h * 32, 16)] = lo
            outc_v[qi, pl.ds(h * 32 + 16, 16)] = hi
            return carry2

        lax.fori_loop(0, _N_HEADS, h_body, 0)

    issue(0, 0)

    def q2_body(qq, carry):
        q0 = qq * 2
        issue(q0 + 1, 1)
        drain(q0, 0)
        compute(q0, 0)

        @pl.when(qq < _NQPW // 2 - 1)
        def _():
            issue(q0 + 2, 0)

        drain(q0 + 1, 1)
        compute(q0 + 1, 1)
        return carry

    lax.fori_loop(0, _NQPW // 2, q2_body, 0)
    pltpu.sync_copy(outc_v, out_hbm.at[pl.ds(base_q, _NQPW)])


def _sc_gather(value2, idx4, wgt4):
    mesh = plsc.VectorSubcoreMesh(core_axis_name="c", subcore_axis_name="s")
    run = functools.partial(
        pl.kernel,
        out_type=jax.ShapeDtypeStruct((_NQ_TOT, _C), jnp.float32),
        mesh=mesh,
        scratch_types=[
            pltpu.VMEM((_NQPW // 8, 4, 8, 128), jnp.int32),
            pltpu.VMEM((_NQPW // 8, 4, 8, 128), jnp.float32),
            pltpu.VMEM((2, 4, 128, 16), jnp.int32),
            pltpu.VMEM((_NQPW, _C), jnp.float32),
            pltpu.SemaphoreType.DMA,
            pltpu.SemaphoreType.DMA,
        ],
        compiler_params=pltpu.CompilerParams(use_tc_tiling_on_sc=False,
                                             needs_layout_passes=False),
    )(_sc_body)
    return run(value2, idx4, wgt4)


def _out_body(x_ref, w_ref, b_ref, o_ref):
    y = jnp.dot(x_ref[...], w_ref[...],
                preferred_element_type=jnp.float32) + b_ref[...]
    o_ref[0] = y[:_LEN_Q]


def _out_proj(acc, Wo, bo2):
    return pl.pallas_call(
        _out_body,
        grid=(_BATCH,),
        in_specs=[
            pl.BlockSpec((_LEN_QP, _C), lambda b: (b, 0)),
            pl.BlockSpec((_C, _C), lambda b: (0, 0)),
            pl.BlockSpec((1, _C), lambda b: (0, 0)),
        ],
        out_specs=pl.BlockSpec((1, _LEN_Q, _C), lambda b: (b, 0, 0)),
        out_shape=jax.ShapeDtypeStruct((_BATCH, _LEN_Q, _C), jnp.float32),
    )(acc, Wo, bo2)


def _col_consts():
    j = np.arange(128)
    l_of = (j // 4) % 4
    h_of = j // 16
    cc = np.zeros((8, 128), dtype=np.float32)
    cc[0] = _SS[l_of, 1].astype(np.float32)
    cc[1] = _SS[l_of, 0].astype(np.float32)
    cc[2] = 1.0 / cc[0]
    cc[3] = 1.0 / cc[1]
    cc[4] = _LS[l_of].astype(np.float32)
    cc[5] = h_of.astype(np.float32)
    return cc


_CC = _col_consts()


def kernel(query, reference_points, inputs, input_spatial_shapes,
           input_level_start_index, Wv, bv, Ws, bs, Wa, ba, Wo, bo):
    inputs2 = inputs.reshape(_BATCH * _LEN_IN, _C)
    qp = jnp.pad(query, ((0, 0), (0, _LEN_QP - _LEN_Q), (0, 0))).reshape(_NQ_TOT, _C)
    rpp = jnp.pad(reference_points,
                  ((0, 0), (0, _LEN_QP - _LEN_Q), (0, 0))).reshape(_NQ_TOT, 2)
    rpx = jnp.broadcast_to(rpp[:, 0:1], (_NQ_TOT, 128))
    rpy = jnp.broadcast_to(rpp[:, 1:2], (_NQ_TOT, 128))
    xy_perm = np.concatenate([np.arange(0, _C, 2), np.arange(1, _C, 2)])
    Ws_p = Ws[:, xy_perm]
    bs_p = bs[xy_perm].reshape(1, _C)
    ba2 = ba.reshape(1, 128)
    bo2 = bo.reshape(1, _C)
    j = np.arange(128)
    cols_lo = (j // 16) * 32 + (j % 16)
    cols_hi = cols_lo + 16
    Wvl, Wvr = Wv[:, cols_lo], Wv[:, cols_hi]
    bvl = bv[cols_lo].reshape(1, 128)
    bvr = bv[cols_hi].reshape(1, 128)

    idx, wgt = _sampling(qp, rpx, rpy, Ws_p, bs_p, Wa, ba2, jnp.asarray(_CC))
    packed = _value_proj(inputs2, Wvl, Wvr, bvl, bvr)
    value2 = packed.reshape(_BATCH * _LEN_IN * _N_HEADS, 16)
    acc = _sc_gather(value2, idx, wgt)
    return _out_proj(acc, Wo, bo2)


# speedup vs baseline: 17.8203x; 1.0341x over previous
import functools

import numpy as np
import jax
import jax.numpy as jnp
from jax import lax
from jax.experimental import pallas as pl
from jax.experimental.pallas import tpu as pltpu
from jax.experimental.pallas import tpu_sc as plsc

_N_HEADS = 8
_C = 256
_BATCH = 4
_LEN_Q = 300
_LEN_QP = 320
_NQ_TOT = _BATCH * _LEN_QP
_SS = np.array([[128, 128], [64, 64], [32, 32], [16, 16]], dtype=np.int64)
_LS = np.array([0, 16384, 20480, 21504], dtype=np.int64)
_LEN_IN = 21760
_NW = 32
_NQPW = _NQ_TOT // _NW


def _rtne_bf16_bits(v):
    bits = lax.bitcast_convert_type(v, jnp.int32)
    r = bits + 0x7FFF + ((bits >> 16) & 1)
    return (r >> 16) & 0xFFFF


def _value_body(x_ref, wl_ref, wr_ref, bl_ref, br_ref, o_ref):
    x = x_ref[...].astype(jnp.bfloat16)
    wl = wl_ref[...].astype(jnp.bfloat16)
    wr = wr_ref[...].astype(jnp.bfloat16)
    lo = jnp.dot(x, wl, preferred_element_type=jnp.float32) + bl_ref[...]
    hi = jnp.dot(x, wr, preferred_element_type=jnp.float32) + br_ref[...]
    o_ref[...] = _rtne_bf16_bits(lo) | (_rtne_bf16_bits(hi) << 16)


def _value_proj(inputs2, Wvl, Wvr, bvl, bvr):
    n = inputs2.shape[0]
    blk = 8704
    grid = n // blk
    return pl.pallas_call(
        _value_body,
        grid=(grid,),
        in_specs=[
            pl.BlockSpec((blk, _C), lambda i: (i, 0)),
            pl.BlockSpec((_C, 128), lambda i: (0, 0)),
            pl.BlockSpec((_C, 128), lambda i: (0, 0)),
            pl.BlockSpec((1, 128), lambda i: (0, 0)),
            pl.BlockSpec((1, 128), lambda i: (0, 0)),
        ],
        out_specs=pl.BlockSpec((blk, 128), lambda i: (i, 0)),
        out_shape=jax.ShapeDtypeStruct((n, 128), jnp.int32),
    )(inputs2, Wvl, Wvr, bvl, bvr)


def _sampling_body(q_ref, rx_ref, ry_ref, ws_ref, bs_ref, wa_ref, ba_ref,
                   cc_ref, idx_ref, wgt_ref):
    q = q_ref[...]
    off = jnp.dot(q, ws_ref[...], preferred_element_type=jnp.float32) + bs_ref[...]
    araw = jnp.dot(q, wa_ref[...], preferred_element_type=jnp.float32) + ba_ref[...]
    parts = []
    for h in range(_N_HEADS):
        a = araw[:, h * 16:(h + 1) * 16]
        m = jnp.max(a, axis=1, keepdims=True)
        e = jnp.exp(a - m)
        parts.append(e / jnp.sum(e, axis=1, keepdims=True))
    attn = jnp.concatenate(parts, axis=1)

    Wf = cc_ref[0:1, :]
    Hf = cc_ref[1:2, :]
    invWf = cc_ref[2:3, :]
    invHf = cc_ref[3:4, :]
    startf = cc_ref[4:5, :]
    hf = cc_ref[5:6, :]

    locx = rx_ref[...] + off[:, :128] * invWf
    locy = ry_ref[...] + off[:, 128:] * invHf
    x = locx * Wf - 0.5
    y = locy * Hf - 0.5
    x0 = jnp.floor(x)
    y0 = jnp.floor(y)
    x1 = x0 + 1.0
    y1 = y0 + 1.0
    wx1 = x - x0
    wx0 = 1.0 - wx1
    wy1 = y - y0
    wy0 = 1.0 - wy1

    Wi = Wf.astype(jnp.int32)
    hi = hf.astype(jnp.int32)
    bi = lax.broadcasted_iota(jnp.int32, (_NQ_TOT, 128), 0) // _LEN_QP
    basei = startf.astype(jnp.int32) + bi * _LEN_IN

    corners = [(x0, y0, wx0 * wy0), (x1, y0, wx1 * wy0),
               (x0, y1, wx0 * wy1), (x1, y1, wx1 * wy1)]
    for c, (xi, yi, wb) in enumerate(corners):
        inb = (xi >= 0.) & (xi <= Wf - 1.) & (yi >= 0.) & (yi <= Hf - 1.)
        xc = jnp.clip(xi, 0., Wf - 1.).astype(jnp.int32)
        yc = jnp.clip(yi, 0., Hf - 1.).astype(jnp.int32)
        rowi = (basei + yc * Wi + xc) * _N_HEADS + hi
        idx_ref[:, c] = rowi.reshape(_NQ_TOT // 8, 8, 128)
        wgt_ref[:, c] = jnp.where(inb, attn * wb, 0.0).reshape(_NQ_TOT // 8, 8, 128)


def _sampling(qp, rpx, rpy, Ws_p, bs_p, Wa, ba2, cc):
    return pl.pallas_call(
        _sampling_body,
        grid=(1,),
        in_specs=[
            pl.BlockSpec((_NQ_TOT, _C), lambda b: (0, 0)),
            pl.BlockSpec((_NQ_TOT, 128), lambda b: (0, 0)),
            pl.BlockSpec((_NQ_TOT, 128), lambda b: (0, 0)),
            pl.BlockSpec((_C, _C), lambda b: (0, 0)),
            pl.BlockSpec((1, _C), lambda b: (0, 0)),
            pl.BlockSpec((_C, 128), lambda b: (0, 0)),
            pl.BlockSpec((1, 128), lambda b: (0, 0)),
            pl.BlockSpec((8, 128), lambda b: (0, 0)),
        ],
        out_specs=[
            pl.BlockSpec((_NQ_TOT // 8, 4, 8, 128), lambda b: (0, 0, 0, 0)),
            pl.BlockSpec((_NQ_TOT // 8, 4, 8, 128), lambda b: (0, 0, 0, 0)),
        ],
        out_shape=[
            jax.ShapeDtypeStruct((_NQ_TOT // 8, 4, 8, 128), jnp.int32),
            jax.ShapeDtypeStruct((_NQ_TOT // 8, 4, 8, 128), jnp.float32),
        ],
    )(qp, rpx, rpy, Ws_p, bs_p, Wa, ba2, cc)


def _sc_body(value_hbm, idx_hbm, wgt_hbm, out_hbm,
             idx_v, wgt_v, rows_v, outc_v, sem0, sem1, sem2, sem3):
    wid = lax.axis_index("s") * 2 + lax.axis_index("c")
    base_q = wid * _NQPW
    base_t = wid * (_NQPW // 8)
    sems = (sem0, sem1, sem2, sem3)
    pltpu.sync_copy(idx_hbm.at[pl.ds(base_t, _NQPW // 8)], idx_v)
    pltpu.sync_copy(wgt_hbm.at[pl.ds(base_t, _NQPW // 8)], wgt_v)

    def issue(qi, slot):
        t, s = qi >> 3, qi & 7
        for c in range(4):
            pltpu.async_copy(
                value_hbm.at[idx_v.at[t, c, s]],
                rows_v.at[slot, c], sems[slot])

    def drain(qi, slot):
        t, s = qi >> 3, qi & 7
        for c in range(4):
            pltpu.make_async_copy(
                value_hbm.at[idx_v.at[t, c, s]],
                rows_v.at[slot, c], sems[slot]).wait()

    def compute(qi, slot):
        t, s = qi >> 3, qi & 7

        def h_body(h, carry2):
            wvs = [wgt_v[t, c, s, pl.ds(h * 16, 16)] for c in range(4)]
            lo = jnp.zeros((16,), jnp.float32)
            hi = jnp.zeros((16,), jnp.float32)
            for j in range(16):
                col = h * 16 + j
                for c in range(4):
                    w = wvs[c][j]
                    w32 = rows_v[slot, c, col, ...]
                    lov = plsc.bitcast(w32 << 16, jnp.float32)
                    hiv = plsc.bitcast(w32, jnp.float32)
                    lo = lo + w * lov
                    hi = hi + w * hiv
            outc_v[qi, pl.ds(h * 32, 16)] = lo
            outc_v[qi, pl.ds(h * 32 + 16, 16)] = hi
            return carry2

        lax.fori_loop(0, _N_HEADS, h_body, 0)

    issue(0, 0)
    issue(1, 1)
    issue(2, 2)

    def q4_body(qq, carry):
        q0 = qq * 4
        issue(q0 + 3, 3)
        drain(q0, 0)
        compute(q0, 0)

        @pl.when(qq < _NQPW // 4 - 1)
        def _():
            issue(q0 + 4, 0)

        drain(q0 + 1, 1)
        compute(q0 + 1, 1)

        @pl.when(qq < _NQPW // 4 - 1)
        def _():
            issue(q0 + 5, 1)

        drain(q0 + 2, 2)
        compute(q0 + 2, 2)

        @pl.when(qq < _NQPW // 4 - 1)
        def _():
            issue(q0 + 6, 2)

        drain(q0 + 3, 3)
        compute(q0 + 3, 3)
        return carry

    lax.fori_loop(0, _NQPW // 4, q4_body, 0)
    pltpu.sync_copy(outc_v, out_hbm.at[pl.ds(base_q, _NQPW)])


def _sc_gather(value2, idx4, wgt4):
    mesh = plsc.VectorSubcoreMesh(core_axis_name="c", subcore_axis_name="s")
    run = functools.partial(
        pl.kernel,
        out_type=jax.ShapeDtypeStruct((_NQ_TOT, _C), jnp.float32),
        mesh=mesh,
        scratch_types=[
            pltpu.VMEM((_NQPW // 8, 4, 8, 128), jnp.int32),
            pltpu.VMEM((_NQPW // 8, 4, 8, 128), jnp.float32),
            pltpu.VMEM((4, 4, 128, 16), jnp.int32),
            pltpu.VMEM((_NQPW, _C), jnp.float32),
            pltpu.SemaphoreType.DMA,
            pltpu.SemaphoreType.DMA,
            pltpu.SemaphoreType.DMA,
            pltpu.SemaphoreType.DMA,
        ],
        compiler_params=pltpu.CompilerParams(use_tc_tiling_on_sc=False,
                                             needs_layout_passes=False),
    )(_sc_body)
    return run(value2, idx4, wgt4)


def _out_body(x_ref, w_ref, b_ref, o_ref):
    y = jnp.dot(x_ref[...], w_ref[...],
                preferred_element_type=jnp.float32) + b_ref[...]
    o_ref[0] = y[:_LEN_Q]


def _out_proj(acc, Wo, bo2):
    return pl.pallas_call(
        _out_body,
        grid=(_BATCH,),
        in_specs=[
            pl.BlockSpec((_LEN_QP, _C), lambda b: (b, 0)),
            pl.BlockSpec((_C, _C), lambda b: (0, 0)),
            pl.BlockSpec((1, _C), lambda b: (0, 0)),
        ],
        out_specs=pl.BlockSpec((1, _LEN_Q, _C), lambda b: (b, 0, 0)),
        out_shape=jax.ShapeDtypeStruct((_BATCH, _LEN_Q, _C), jnp.float32),
    )(acc, Wo, bo2)


def _col_consts():
    j = np.arange(128)
    l_of = (j // 4) % 4
    h_of = j // 16
    cc = np.zeros((8, 128), dtype=np.float32)
    cc[0] = _SS[l_of, 1].astype(np.float32)
    cc[1] = _SS[l_of, 0].astype(np.float32)
    cc[2] = 1.0 / cc[0]
    cc[3] = 1.0 / cc[1]
    cc[4] = _LS[l_of].astype(np.float32)
    cc[5] = h_of.astype(np.float32)
    return cc


_CC = _col_consts()


def kernel(query, reference_points, inputs, input_spatial_shapes,
           input_level_start_index, Wv, bv, Ws, bs, Wa, ba, Wo, bo):
    inputs2 = inputs.reshape(_BATCH * _LEN_IN, _C)
    qp = jnp.pad(query, ((0, 0), (0, _LEN_QP - _LEN_Q), (0, 0))).reshape(_NQ_TOT, _C)
    rpp = jnp.pad(reference_points,
                  ((0, 0), (0, _LEN_QP - _LEN_Q), (0, 0))).reshape(_NQ_TOT, 2)
    rpx = jnp.broadcast_to(rpp[:, 0:1], (_NQ_TOT, 128))
    rpy = jnp.broadcast_to(rpp[:, 1:2], (_NQ_TOT, 128))
    xy_perm = np.concatenate([np.arange(0, _C, 2), np.arange(1, _C, 2)])
    Ws_p = Ws[:, xy_perm]
    bs_p = bs[xy_perm].reshape(1, _C)
    ba2 = ba.reshape(1, 128)
    bo2 = bo.reshape(1, _C)
    j = np.arange(128)
    cols_lo = (j // 16) * 32 + (j % 16)
    cols_hi = cols_lo + 16
    Wvl, Wvr = Wv[:, cols_lo], Wv[:, cols_hi]
    bvl = bv[cols_lo].reshape(1, 128)
    bvr = bv[cols_hi].reshape(1, 128)

    idx, wgt = _sampling(qp, rpx, rpy, Ws_p, bs_p, Wa, ba2, jnp.asarray(_CC))
    packed = _value_proj(inputs2, Wvl, Wvr, bvl, bvr)
    value2 = packed.reshape(_BATCH * _LEN_IN * _N_HEADS, 16)
    acc = _sc_gather(value2, idx, wgt)
    return _out_proj(acc, Wo, bo2)
